# baseline (speedup 1.0000x reference)
import functools

import numpy as np
import jax
import jax.numpy as jnp
from jax import lax
from jax.experimental import pallas as pl
from jax.experimental.pallas import tpu as pltpu
from jax.experimental.pallas import tpu_sc as plsc

_L = 512
_K = 48
_H = 128
_E = _L * _K
_NW = 32


def _sc_gather(table, idx, *, n_bufs=8, sub=48):
    n_rows = idx.shape[0]
    v, d = table.shape
    b_per_w = n_rows // _NW
    outer = n_bufs * sub
    n_outer = b_per_w // outer
    assert b_per_w % outer == 0 and sub % 8 == 0
    assert n_rows % (8 * _NW) == 0

    stage = min(v, 256)
    mesh = plsc.VectorSubcoreMesh(core_axis_name="c", subcore_axis_name="s")

    @functools.partial(
        pl.kernel,
        mesh=mesh,
        out_type=jax.ShapeDtypeStruct((n_rows, d), jnp.float32),
        scratch_types=[
            pltpu.VMEM((b_per_w,), jnp.int32),
            pltpu.VMEM((n_bufs, sub, d), jnp.float32),
            pltpu.VMEM((stage, d), jnp.float32),
            pltpu.VMEM_SHARED((v, d), jnp.float32),
            pltpu.SemaphoreType.DMA,
        ],
    )
    def gk(table_hbm, idx_hbm, out_hbm, idx_v, rows_v, stage_v, shared_v,
           sem):
        sid = lax.axis_index("s")
        wid = sid * 2 + lax.axis_index("c")
        base = wid * b_per_w
        pltpu.sync_copy(idx_hbm.at[pl.ds(base, b_per_w)], idx_v)

        @pl.when(sid == 0)
        def _():
            off = 0
            while off < v:
                c = min(stage, v - off)
                pltpu.sync_copy(table_hbm.at[pl.ds(off, c)],
                                stage_v.at[pl.ds(0, c)])
                pltpu.sync_copy(stage_v.at[pl.ds(0, c)],
                                shared_v.at[pl.ds(off, c)])
                off += c

        plsc.subcore_barrier()
        for oc in range(n_outer):
            copies = [
                pltpu.async_copy(
                    shared_v.at[idx_v.at[pl.ds(oc * outer + j * sub, sub)]],
                    rows_v.at[j], sem)
                for j in range(n_bufs)
            ]
            for j in range(n_bufs):
                copies[j].wait()
                pltpu.sync_copy(
                    rows_v.at[j],
                    out_hbm.at[pl.ds(base + oc * outer + j * sub, sub)])

    return gk(table, idx)


def _k_graph(ca_ref, cat_ref, n_ref, c_ref, o_ref, s1_ref, s2_ref,
             pca, pcn, pcc, pco, pcb, eidx_ref, atoms_ref):
    ca = ca_ref[...]
    cat = cat_ref[...]
    g = jnp.dot(ca, cat)
    ncol = jnp.sum(ca * ca, axis=1, keepdims=True)
    nrow = jnp.sum(cat * cat, axis=0, keepdims=True)
    dist = jnp.sqrt(jnp.maximum(ncol + nrow - 2.0 * g, 0.0) + 1e-6)
    iota = lax.broadcasted_iota(jnp.int32, (_L, _L), 1).astype(jnp.float32)

    def step(k, carry):
        dmat, eidx = carry
        mn = jnp.min(dmat, axis=1, keepdims=True)
        idxf = jnp.min(jnp.where(dmat <= mn, iota, jnp.float32(_L)),
                       axis=1, keepdims=True)
        kcol = lax.broadcasted_iota(jnp.int32, (_L, _K), 1)
        eidx = jnp.where(kcol == k, idxf.astype(jnp.int32), eidx)
        dmat = jnp.where(iota == idxf, jnp.float32(jnp.inf), dmat)
        return dmat, eidx

    _, eidx = lax.fori_loop(0, _K, step,
                            (dist, jnp.zeros((_L, _K), jnp.int32)))
    eidx_ref[...] = eidx

    nn = n_ref[...]
    cc = c_ref[...]
    oo = o_ref[...]
    b = ca - nn
    c2 = cc - ca
    s1 = s1_ref[...]
    s2 = s2_ref[...]
    a = jnp.dot(b, s1) * jnp.dot(c2, s2) - jnp.dot(b, s2) * jnp.dot(c2, s1)
    cb = -0.58273431 * a + 0.56802827 * b - 0.54067466 * c2 + ca
    atoms_ref[...] = (jnp.dot(ca, pca[...]) + jnp.dot(nn, pcn[...]) +
                      jnp.dot(cc, pcc[...]) + jnp.dot(oo, pco[...]) +
                      jnp.dot(cb, pcb[...]))


def _ln(x):
    m = jnp.mean(x, -1, keepdims=True)
    xm = x - m
    v = jnp.mean(xm * xm, -1, keepdims=True)
    return xm / jnp.sqrt(v + 1e-5)


def _bdot(a, b):
    return jnp.dot(a.astype(jnp.bfloat16), b.astype(jnp.bfloat16),
                   preferred_element_type=jnp.float32)


_TEB = 512


def _k_feat(nb_ref, qc_ref, pos_ref, a1, a2, p1, p2, a3, b1,
            bpos, wep, wer, be, out_ref):
    q = qc_ref[...]
    n = nb_ref[...]
    q2 = jnp.dot(q * q, a1[...])
    n2 = jnp.dot(n * n, a2[...])
    x = jnp.dot(q, p1[...]) * jnp.dot(n, p2[...])
    cross = jnp.dot(x, a3[...])
    d = jnp.sqrt(jnp.maximum(q2 + n2 - 2.0 * cross, 0.0) + 1e-6)
    db = jnp.dot(d, b1[...])
    lane = lax.broadcasted_iota(jnp.int32, (_TEB, 400), 1)
    mu = 2.0 + (4.0 / 3.0) * (lane % 16).astype(jnp.float32)
    rbf = jnp.exp(-(((db - mu) / 1.25) ** 2))
    ep = pos_ref[...] + bpos[0:1, :]
    pre = _bdot(ep, wep[...]) + _bdot(rbf, wer[...]) + be[0:1, :]
    out_ref[...] = _ln(pre)


_TN = 64
_TE = _TN * _K


def _rmask():
    r = (lax.broadcasted_iota(jnp.int32, (_TE, _TN), 0) // _K ==
         lax.broadcasted_iota(jnp.int32, (_TE, _TN), 1))
    return r.astype(jnp.float32)


def _rtmask():
    r = (lax.broadcasted_iota(jnp.int32, (_TN, _TE), 1) // _K ==
         lax.broadcasted_iota(jnp.int32, (_TN, _TE), 0))
    return r.astype(jnp.float32)


def _node_tail(hv, msum, win, wout, out_ref):
    h1 = _ln(hv + msum / 30.0)
    f = _bdot(jax.nn.gelu(_bdot(h1, win[...])), wout[...])
    out_ref[...] = _ln(h1 + f)


def _k_encnode_first(he_ref, w1b, w2, w3, win, wout, out_ref):
    m = jax.nn.gelu(_bdot(he_ref[...], w1b[...]))
    m = jax.nn.gelu(_bdot(m, w2[...]))
    m = _bdot(m, w3[...])
    msum = _bdot(_rtmask(), m)
    _node_tail(jnp.zeros((_TN, _H), jnp.float32), msum, win, wout, out_ref)


def _k_encnode(hv_ref, he_ref, vn_ref, w1a, w1b, w1c, w2, w3, win, wout,
               out_ref):
    hv = hv_ref[...]
    m1 = (_bdot(_rmask(), _bdot(hv, w1a[...])) +
          _bdot(he_ref[...], w1b[...]) + _bdot(vn_ref[...], w1c[...]))
    m = jax.nn.gelu(m1)
    m = jax.nn.gelu(_bdot(m, w2[...]))
    m = _bdot(m, w3[...])
    msum = _bdot(_rtmask(), m)
    _node_tail(hv, msum, win, wout, out_ref)


def _k_encedge(hv_ref, he_ref, vn_ref, w1a, w1b, w1c, w2, w3, out_ref):
    he = he_ref[...]
    m1 = (_bdot(_rmask(), _bdot(hv_ref[...], w1a[...])) +
          _bdot(he, w1b[...]) + _bdot(vn_ref[...], w1c[...]))
    m = jax.nn.gelu(m1)
    m = jax.nn.gelu(_bdot(m, w2[...]))
    m = _bdot(m, w3[...])
    out_ref[...] = _ln(he + m)


def _k_dec(hvd_ref, he_ref, ms_ref, fv_ref, mb_ref,
           w1a, w1b, w1c, w1d, w2, w3, win, wout, out_ref):
    hvd = hvd_ref[...]
    m1 = (_bdot(_rmask(), _bdot(hvd, w1a[...])) +
          _bdot(he_ref[...], w1b[...]) + _bdot(ms_ref[...], w1c[...]) +
          _bdot(fv_ref[...] + mb_ref[...], w1d[...]))
    m = jax.nn.gelu(m1)
    m = jax.nn.gelu(_bdot(m, w2[...]))
    m = _bdot(m, w3[...])
    msum = _bdot(_rtmask(), m)
    _node_tail(hvd, msum, win, wout, out_ref)


def _k_final(t_ref, hvd_ref, wo_ref, bo_ref, out_ref):
    tt = t_ref[0]
    r0 = hvd_ref[pl.ds(tt, 1), :]
    r1 = hvd_ref[pl.ds(tt + _L, 1), :]
    rows = jnp.concatenate([r0, r1], 0)
    lg = jnp.dot(rows, wo_ref[...]) + bo_ref[0:1, :]
    mx = jnp.max(lg, -1, keepdims=True)
    e = jnp.exp(lg - mx)
    p = e / jnp.sum(e, -1, keepdims=True)
    out_ref[...] = p[:, :20]


def _consts():
    a1 = np.zeros((128, 25), np.float32)
    a2 = np.zeros((128, 25), np.float32)
    p1 = np.zeros((128, 75), np.float32)
    p2 = np.zeros((128, 75), np.float32)
    a3 = np.zeros((75, 25), np.float32)
    b1 = np.zeros((25, 400), np.float32)
    for p in range(25):
        a, b = p // 5, p % 5
        for c in range(3):
            a1[3 * a + c, p] = 1.0
            a2[3 * b + c, p] = 1.0
            p1[3 * a + c, 3 * p + c] = 1.0
            p2[3 * b + c, 3 * p + c] = 1.0
            a3[3 * p + c, p] = 1.0
        b1[p, 16 * p:16 * (p + 1)] = 1.0
    s1 = np.zeros((8, 8), np.float32)
    s2 = np.zeros((8, 8), np.float32)
    for j in range(3):
        s1[(j + 1) % 3, j] = 1.0
        s2[(j + 2) % 3, j] = 1.0
    packs = []
    for ai in range(5):
        pk = np.zeros((8, 16), np.float32)
        for c in range(3):
            pk[c, 3 * ai + c] = 1.0
        packs.append(pk)
    return ([jnp.asarray(m) for m in (a1, a2, p1, p2, a3, b1, s1, s2)],
            [jnp.asarray(m) for m in packs])


def _full_spec(shape):
    return pl.BlockSpec(shape, lambda g: tuple(0 for _ in shape))


def kernel(struct, seq, decode_order, token_to_decode, params):
    f32, i32 = jnp.float32, jnp.int32
    struct = struct.astype(f32)
    nn, ca, cc, oo = (struct[:, 0, :], struct[:, 1, :],
                      struct[:, 2, :], struct[:, 3, :])
    pad8 = lambda x: jnp.pad(x, ((0, 0), (0, 5)))
    ca8, n8, c8, o8 = pad8(ca), pad8(nn), pad8(cc), pad8(oo)
    cat = ca8.T
    (a1, a2, p1, p2, a3, b1, s1, s2), packs = _consts()

    eidx, atoms16 = pl.pallas_call(
        _k_graph,
        out_shape=[jax.ShapeDtypeStruct((_L, _K), i32),
                   jax.ShapeDtypeStruct((_L, 16), f32)],
    )(ca8, cat, n8, c8, o8, s1, s2, *packs)

    nbidx = eidx.reshape(-1).astype(i32)
    qidx = jnp.repeat(jnp.arange(_L, dtype=i32), _K)
    posidx = jnp.clip(qidx - nbidx + 32, 0, 64) + _L
    feat_idx = jnp.concatenate([nbidx, qidx, posidx])

    wpos = params['W_pos'].astype(f32)
    table_feat = jnp.pad(jnp.concatenate([atoms16, wpos], 0),
                         ((0, 0), (0, 112)))
    gf = _sc_gather(table_feat, feat_idx)
    nbq, qcq, posq = gf[:_E], gf[_E:2 * _E], gf[2 * _E:]

    bpos = jnp.tile(jnp.pad(params['b_pos'].astype(f32), (0, 112))[None, :],
                    (8, 1))
    be = jnp.tile(params['b_e'].astype(f32)[None, :], (8, 1))
    wep = jnp.pad(params['W_e'][:16].astype(f32), ((0, 112), (0, 0)))
    wer = params['W_e'][16:].astype(f32)
    n_ft = _E // _TEB
    he = pl.pallas_call(
        _k_feat,
        grid=(n_ft,),
        in_specs=[pl.BlockSpec((_TEB, _H), lambda g: (g, 0)),
                  pl.BlockSpec((_TEB, _H), lambda g: (g, 0)),
                  pl.BlockSpec((_TEB, _H), lambda g: (g, 0)),
                  _full_spec((_H, 25)), _full_spec((_H, 25)),
                  _full_spec((_H, 75)), _full_spec((_H, 75)),
                  _full_spec((75, 25)), _full_spec((25, 400)),
                  _full_spec((8, _H)), _full_spec((_H, _H)),
                  _full_spec((400, _H)), _full_spec((8, _H))],
        out_specs=pl.BlockSpec((_TEB, _H), lambda g: (g, 0)),
        out_shape=jax.ShapeDtypeStruct((_E, _H), f32),
    )(nbq, qcq, posq, a1, a2, p1, p2, a3, b1, bpos, wep, wer, be)

    ngrid = _L // _TN
    nspec = pl.BlockSpec((_TN, _H), lambda g: (g, 0))
    espec = pl.BlockSpec((_TE, _H), lambda g: (g, 0))
    w128 = _full_spec((_H, _H))
    wi = _full_spec((_H, 4 * _H))
    wo = _full_spec((4 * _H, _H))

    hv = None
    vn = None
    for li, lp in enumerate(params['enc']):
        w1a, w1b, w1c = (lp['W1'][:_H], lp['W1'][_H:2 * _H],
                         lp['W1'][2 * _H:])
        if li == 0:
            hv = pl.pallas_call(
                _k_encnode_first, grid=(ngrid,),
                in_specs=[espec, w128, w128, w128, wi, wo],
                out_specs=nspec,
                out_shape=jax.ShapeDtypeStruct((_L, _H), f32),
            )(he, w1b, lp['W2'], lp['W3'], lp['Win'], lp['Wout'])
        else:
            hv = pl.pallas_call(
                _k_encnode, grid=(ngrid,),
                in_specs=[nspec, espec, espec, w128, w128, w128, w128, w128,
                          wi, wo],
                out_specs=nspec,
                out_shape=jax.ShapeDtypeStruct((_L, _H), f32),
            )(hv, he, vn, w1a, w1b, w1c, lp['W2'], lp['W3'],
              lp['Win'], lp['Wout'])
        vn = _sc_gather(hv, nbidx)
        w11a, w11b, w11c = (lp['W11'][:_H], lp['W11'][_H:2 * _H],
                            lp['W11'][2 * _H:])
        he = pl.pallas_call(
            _k_encedge, grid=(ngrid,),
            in_specs=[nspec, espec, espec, w128, w128, w128, w128, w128],
            out_specs=espec,
            out_shape=jax.ShapeDtypeStruct((_E, _H), f32),
        )(hv, he, vn, w11a, w11b, w11c, lp['W12'], lp['W13'])

    seqi = seq.astype(i32)
    rank = jnp.zeros(_L, i32).at[decode_order].set(jnp.arange(_L, dtype=i32))
    bw = jnp.take(rank, nbidx) < jnp.take(rank, qidx)
    seqnb = jnp.take(seqi, nbidx, axis=1)
    ms_idx = jnp.where(bw[None, :], seqnb, 21)
    fv_idx = jnp.where(bw, 21, nbidx + 22)
    ds_idx = jnp.concatenate([ms_idx.reshape(-1), fv_idx]).astype(i32)
    table_ds = jnp.concatenate(
        [params['W_s'].astype(f32), jnp.zeros((1, _H), f32), hv], 0)
    ds = _sc_gather(table_ds, ds_idx)
    ms, fv = ds[:2 * _E], ds[2 * _E:]

    boff = jnp.array([[0], [_L]], i32)
    dec_idx = jnp.where(bw[None, :], nbidx[None, :] + boff,
                        2 * _L).reshape(-1).astype(i32)

    hvd = jnp.concatenate([hv, hv], 0)
    zrow = jnp.zeros((1, _H), f32)
    dgrid = 2 * ngrid
    nspec_d = pl.BlockSpec((_TN, _H), lambda g: (g, 0))
    espec_d = pl.BlockSpec((_TE, _H), lambda g: (g, 0))
    espec_s = pl.BlockSpec((_TE, _H), lambda g: (g % ngrid, 0))
    for lp in params['dec']:
        mb = _sc_gather(jnp.concatenate([hvd, zrow], 0), dec_idx)
        w1a, w1b, w1c, w1d = (lp['W1'][:_H], lp['W1'][_H:2 * _H],
                              lp['W1'][2 * _H:3 * _H], lp['W1'][3 * _H:])
        hvd = pl.pallas_call(
            _k_dec, grid=(dgrid,),
            in_specs=[nspec_d, espec_s, espec_d, espec_s, espec_d,
                      w128, w128, w128, w128, w128, w128, wi, wo],
            out_specs=nspec_d,
            out_shape=jax.ShapeDtypeStruct((2 * _L, _H), f32),
        )(hvd, he, ms, fv, mb, w1a, w1b, w1c, w1d, lp['W2'], lp['W3'],
          lp['Win'], lp['Wout'])

    t_arr = jnp.asarray(token_to_decode, i32).reshape(1)
    bo = params['b_o'].astype(f32).reshape(1, 21)
    probs = pl.pallas_call(
        _k_final,
        in_specs=[pl.BlockSpec(memory_space=pltpu.SMEM),
                  pl.BlockSpec(memory_space=pltpu.VMEM),
                  pl.BlockSpec(memory_space=pltpu.VMEM),
                  pl.BlockSpec(memory_space=pltpu.VMEM)],
        out_shape=jax.ShapeDtypeStruct((2, 20), f32),
    )(t_arr, hvd, params['W_o'].astype(f32), bo)
    return probs


# speedup vs baseline: 1.1288x; 1.1288x over previous
import functools

import numpy as np
import jax
import jax.numpy as jnp
from jax import lax
from jax.experimental import pallas as pl
from jax.experimental.pallas import tpu as pltpu
from jax.experimental.pallas import tpu_sc as plsc

_L = 512
_K = 48
_H = 128
_E = _L * _K
_NW = 32


def _sc_gather(table, idx, *, n_bufs=8, sub=48):
    n_rows = idx.shape[0]
    v, d = table.shape
    b_per_w = n_rows // _NW
    outer = n_bufs * sub
    n_outer = b_per_w // outer
    assert b_per_w % outer == 0 and sub % 8 == 0
    assert n_rows % (8 * _NW) == 0

    stage = min(v, 256)
    mesh = plsc.VectorSubcoreMesh(core_axis_name="c", subcore_axis_name="s")

    @functools.partial(
        pl.kernel,
        mesh=mesh,
        out_type=jax.ShapeDtypeStruct((n_rows, d), jnp.float32),
        scratch_types=[
            pltpu.VMEM((b_per_w,), jnp.int32),
            pltpu.VMEM((n_bufs, sub, d), jnp.float32),
            pltpu.VMEM((stage, d), jnp.float32),
            pltpu.VMEM_SHARED((v, d), jnp.float32),
            pltpu.SemaphoreType.DMA,
        ],
    )
    def gk(table_hbm, idx_hbm, out_hbm, idx_v, rows_v, stage_v, shared_v,
           sem):
        sid = lax.axis_index("s")
        wid = sid * 2 + lax.axis_index("c")
        base = wid * b_per_w
        pltpu.sync_copy(idx_hbm.at[pl.ds(base, b_per_w)], idx_v)

        @pl.when(sid == 0)
        def _():
            off = 0
            while off < v:
                c = min(stage, v - off)
                pltpu.sync_copy(table_hbm.at[pl.ds(off, c)],
                                stage_v.at[pl.ds(0, c)])
                pltpu.sync_copy(stage_v.at[pl.ds(0, c)],
                                shared_v.at[pl.ds(off, c)])
                off += c

        plsc.subcore_barrier()
        for oc in range(n_outer):
            copies = [
                pltpu.async_copy(
                    shared_v.at[idx_v.at[pl.ds(oc * outer + j * sub, sub)]],
                    rows_v.at[j], sem)
                for j in range(n_bufs)
            ]
            for j in range(n_bufs):
                copies[j].wait()
                pltpu.sync_copy(
                    rows_v.at[j],
                    out_hbm.at[pl.ds(base + oc * outer + j * sub, sub)])

    return gk(table, idx)


def _k_graph(ca_ref, cat_ref, n_ref, c_ref, o_ref, s1_ref, s2_ref,
             pca, pcn, pcc, pco, pcb, eidx_ref, atoms_ref):
    ca = ca_ref[...]
    cat = cat_ref[...]
    g = jnp.dot(ca, cat)
    ncol = jnp.sum(ca * ca, axis=1, keepdims=True)
    nrow = jnp.sum(cat * cat, axis=0, keepdims=True)
    dist = jnp.sqrt(jnp.maximum(ncol + nrow - 2.0 * g, 0.0) + 1e-6)
    iota = lax.broadcasted_iota(jnp.int32, (_L, _L), 1).astype(jnp.float32)

    def step(k, carry):
        dmat, eidx = carry
        mn = jnp.min(dmat, axis=1, keepdims=True)
        idxf = jnp.min(jnp.where(dmat <= mn, iota, jnp.float32(_L)),
                       axis=1, keepdims=True)
        kcol = lax.broadcasted_iota(jnp.int32, (_L, _K), 1)
        eidx = jnp.where(kcol == k, idxf.astype(jnp.int32), eidx)
        dmat = jnp.where(iota == idxf, jnp.float32(jnp.inf), dmat)
        return dmat, eidx

    _, eidx = lax.fori_loop(0, _K, step,
                            (dist, jnp.zeros((_L, _K), jnp.int32)))
    eidx_ref[...] = eidx

    nn = n_ref[...]
    cc = c_ref[...]
    oo = o_ref[...]
    b = ca - nn
    c2 = cc - ca
    s1 = s1_ref[...]
    s2 = s2_ref[...]
    a = jnp.dot(b, s1) * jnp.dot(c2, s2) - jnp.dot(b, s2) * jnp.dot(c2, s1)
    cb = -0.58273431 * a + 0.56802827 * b - 0.54067466 * c2 + ca
    atoms_ref[...] = (jnp.dot(ca, pca[...]) + jnp.dot(nn, pcn[...]) +
                      jnp.dot(cc, pcc[...]) + jnp.dot(oo, pco[...]) +
                      jnp.dot(cb, pcb[...]))


def _ln(x):
    m = jnp.mean(x, -1, keepdims=True)
    xm = x - m
    v = jnp.mean(xm * xm, -1, keepdims=True)
    return xm / jnp.sqrt(v + 1e-5)


def _bdot(a, b):
    return jnp.dot(a, b, preferred_element_type=jnp.float32)


_TEB = 512


def _k_feat(nb_ref, qc_ref, pos_ref, a1, a2, p1, p2, a3, b1,
            bpos, wep, wer, be, out_ref):
    q = qc_ref[...]
    n = nb_ref[...]
    q2 = jnp.dot(q * q, a1[...])
    n2 = jnp.dot(n * n, a2[...])
    x = jnp.dot(q, p1[...]) * jnp.dot(n, p2[...])
    cross = jnp.dot(x, a3[...])
    d = jnp.sqrt(jnp.maximum(q2 + n2 - 2.0 * cross, 0.0) + 1e-6)
    db = jnp.dot(d, b1[...])
    lane = lax.broadcasted_iota(jnp.int32, (_TEB, 400), 1)
    mu = 2.0 + (4.0 / 3.0) * (lane % 16).astype(jnp.float32)
    rbf = jnp.exp(-(((db - mu) / 1.25) ** 2))
    ep = pos_ref[...] + bpos[0:1, :]
    pre = _bdot(ep, wep[...]) + _bdot(rbf, wer[...]) + be[0:1, :]
    out_ref[...] = _ln(pre)


_TN = 64
_TE = _TN * _K


def _rmask():
    r = (lax.broadcasted_iota(jnp.int32, (_TE, _TN), 0) // _K ==
         lax.broadcasted_iota(jnp.int32, (_TE, _TN), 1))
    return r.astype(jnp.float32)


def _rtmask():
    r = (lax.broadcasted_iota(jnp.int32, (_TN, _TE), 1) // _K ==
         lax.broadcasted_iota(jnp.int32, (_TN, _TE), 0))
    return r.astype(jnp.float32)


def _node_tail(hv, msum, win, wout, out_ref):
    h1 = _ln(hv + msum / 30.0)
    f = _bdot(jax.nn.gelu(_bdot(h1, win[...])), wout[...])
    out_ref[...] = _ln(h1 + f)


def _k_encnode_first(he_ref, w1b, w2, w3, win, wout, out_ref):
    m = jax.nn.gelu(_bdot(he_ref[...], w1b[...]))
    m = jax.nn.gelu(_bdot(m, w2[...]))
    m = _bdot(m, w3[...])
    msum = _bdot(_rtmask(), m)
    _node_tail(jnp.zeros((_TN, _H), jnp.float32), msum, win, wout, out_ref)


def _k_encnode(hv_ref, he_ref, vn_ref, w1a, w1b, w1c, w2, w3, win, wout,
               out_ref):
    hv = hv_ref[...]
    m1 = (_bdot(_rmask(), _bdot(hv, w1a[...])) +
          _bdot(he_ref[...], w1b[...]) + _bdot(vn_ref[...], w1c[...]))
    m = jax.nn.gelu(m1)
    m = jax.nn.gelu(_bdot(m, w2[...]))
    m = _bdot(m, w3[...])
    msum = _bdot(_rtmask(), m)
    _node_tail(hv, msum, win, wout, out_ref)


def _k_encedge(hv_ref, he_ref, vn_ref, w1a, w1b, w1c, w2, w3, out_ref):
    he = he_ref[...]
    m1 = (_bdot(_rmask(), _bdot(hv_ref[...], w1a[...])) +
          _bdot(he, w1b[...]) + _bdot(vn_ref[...], w1c[...]))
    m = jax.nn.gelu(m1)
    m = jax.nn.gelu(_bdot(m, w2[...]))
    m = _bdot(m, w3[...])
    out_ref[...] = _ln(he + m)


def _k_dec(hvd_ref, he_ref, ms_ref, fv_ref, mb_ref,
           w1a, w1b, w1c, w1d, w2, w3, win, wout, out_ref):
    hvd = hvd_ref[...]
    m1 = (_bdot(_rmask(), _bdot(hvd, w1a[...])) +
          _bdot(he_ref[...], w1b[...]) + _bdot(ms_ref[...], w1c[...]) +
          _bdot(fv_ref[...] + mb_ref[...], w1d[...]))
    m = jax.nn.gelu(m1)
    m = jax.nn.gelu(_bdot(m, w2[...]))
    m = _bdot(m, w3[...])
    msum = _bdot(_rtmask(), m)
    _node_tail(hvd, msum, win, wout, out_ref)


def _k_dec3(t_ref, hvd_ref, oh_ref, he2_ref, ms2_ref, fv2_ref,
            w1a, w1b, w1c, w1d, w2, w3, win, wout, wo_ref, bo_ref, out_ref):
    tt = t_ref[0]
    hvd = hvd_ref[...]
    r0 = hvd_ref[pl.ds(tt, 1), :]
    r1 = hvd_ref[pl.ds(tt + _L, 1), :]
    hvt = jnp.concatenate([r0, r1], 0)
    mb = _bdot(oh_ref[...], hvd)
    rm = (lax.broadcasted_iota(jnp.int32, (2 * _K, 2), 0) // _K ==
          lax.broadcasted_iota(jnp.int32, (2 * _K, 2), 1)).astype(jnp.float32)
    rt = (lax.broadcasted_iota(jnp.int32, (2, 2 * _K), 1) // _K ==
          lax.broadcasted_iota(jnp.int32, (2, 2 * _K), 0)).astype(jnp.float32)
    m1 = (_bdot(rm, _bdot(hvt, w1a[...])) + _bdot(he2_ref[...], w1b[...]) +
          _bdot(ms2_ref[...], w1c[...]) +
          _bdot(fv2_ref[...] + mb, w1d[...]))
    m = jax.nn.gelu(m1)
    m = jax.nn.gelu(_bdot(m, w2[...]))
    m = _bdot(m, w3[...])
    h1 = _ln(hvt + _bdot(rt, m) / 30.0)
    f = _bdot(jax.nn.gelu(_bdot(h1, win[...])), wout[...])
    h3 = _ln(h1 + f)
    lg = jnp.dot(h3, wo_ref[...]) + bo_ref[0:1, :]
    mx = jnp.max(lg, -1, keepdims=True)
    e = jnp.exp(lg - mx)
    p = e / jnp.sum(e, -1, keepdims=True)
    out_ref[...] = p[:, :20]


def _consts():
    a1 = np.zeros((128, 25), np.float32)
    a2 = np.zeros((128, 25), np.float32)
    p1 = np.zeros((128, 75), np.float32)
    p2 = np.zeros((128, 75), np.float32)
    a3 = np.zeros((75, 25), np.float32)
    b1 = np.zeros((25, 400), np.float32)
    for p in range(25):
        a, b = p // 5, p % 5
        for c in range(3):
            a1[3 * a + c, p] = 1.0
            a2[3 * b + c, p] = 1.0
            p1[3 * a + c, 3 * p + c] = 1.0
            p2[3 * b + c, 3 * p + c] = 1.0
            a3[3 * p + c, p] = 1.0
        b1[p, 16 * p:16 * (p + 1)] = 1.0
    s1 = np.zeros((8, 8), np.float32)
    s2 = np.zeros((8, 8), np.float32)
    for j in range(3):
        s1[(j + 1) % 3, j] = 1.0
        s2[(j + 2) % 3, j] = 1.0
    packs = []
    for ai in range(5):
        pk = np.zeros((8, 16), np.float32)
        for c in range(3):
            pk[c, 3 * ai + c] = 1.0
        packs.append(pk)
    return ([jnp.asarray(m) for m in (a1, a2, p1, p2, a3, b1, s1, s2)],
            [jnp.asarray(m) for m in packs])


def _full_spec(shape):
    return pl.BlockSpec(shape, lambda g: tuple(0 for _ in shape))


def kernel(struct, seq, decode_order, token_to_decode, params):
    f32, i32 = jnp.float32, jnp.int32
    struct = struct.astype(f32)
    nn, ca, cc, oo = (struct[:, 0, :], struct[:, 1, :],
                      struct[:, 2, :], struct[:, 3, :])
    pad8 = lambda x: jnp.pad(x, ((0, 0), (0, 5)))
    ca8, n8, c8, o8 = pad8(ca), pad8(nn), pad8(cc), pad8(oo)
    cat = ca8.T
    (a1, a2, p1, p2, a3, b1, s1, s2), packs = _consts()

    eidx, atoms16 = pl.pallas_call(
        _k_graph,
        out_shape=[jax.ShapeDtypeStruct((_L, _K), i32),
                   jax.ShapeDtypeStruct((_L, 16), f32)],
    )(ca8, cat, n8, c8, o8, s1, s2, *packs)

    nbidx = eidx.reshape(-1).astype(i32)
    qidx = jnp.repeat(jnp.arange(_L, dtype=i32), _K)
    posidx = jnp.clip(qidx - nbidx + 32, 0, 64) + _L
    feat_idx = jnp.concatenate([nbidx, qidx, posidx])

    wpos = params['W_pos'].astype(f32)
    table_feat = jnp.pad(jnp.concatenate([atoms16, wpos], 0),
                         ((0, 0), (0, 112)))
    gf = _sc_gather(table_feat, feat_idx)

    bpos = jnp.tile(jnp.pad(params['b_pos'].astype(f32), (0, 112))[None, :],
                    (8, 1))
    be = jnp.tile(params['b_e'].astype(f32)[None, :], (8, 1))
    wep = jnp.pad(params['W_e'][:16].astype(f32), ((0, 112), (0, 0)))
    wer = params['W_e'][16:].astype(f32)
    n_ft = _E // _TEB
    he = pl.pallas_call(
        _k_feat,
        grid=(n_ft,),
        in_specs=[pl.BlockSpec((_TEB, _H), lambda g: (g, 0)),
                  pl.BlockSpec((_TEB, _H), lambda g: (48 + g, 0)),
                  pl.BlockSpec((_TEB, _H), lambda g: (96 + g, 0)),
                  _full_spec((_H, 25)), _full_spec((_H, 25)),
                  _full_spec((_H, 75)), _full_spec((_H, 75)),
                  _full_spec((75, 25)), _full_spec((25, 400)),
                  _full_spec((8, _H)), _full_spec((_H, _H)),
                  _full_spec((400, _H)), _full_spec((8, _H))],
        out_specs=pl.BlockSpec((_TEB, _H), lambda g: (g, 0)),
        out_shape=jax.ShapeDtypeStruct((_E, _H), f32),
    )(gf, gf, gf, a1, a2, p1, p2, a3, b1, bpos, wep, wer, be)

    ngrid = _L // _TN
    nspec = pl.BlockSpec((_TN, _H), lambda g: (g, 0))
    espec = pl.BlockSpec((_TE, _H), lambda g: (g, 0))
    w128 = _full_spec((_H, _H))
    wi = _full_spec((_H, 4 * _H))
    wo = _full_spec((4 * _H, _H))

    hv = None
    vn = None
    for li, lp in enumerate(params['enc']):
        w1a, w1b, w1c = (lp['W1'][:_H], lp['W1'][_H:2 * _H],
                         lp['W1'][2 * _H:])
        if li == 0:
            hv = pl.pallas_call(
                _k_encnode_first, grid=(ngrid,),
                in_specs=[espec, w128, w128, w128, wi, wo],
                out_specs=nspec,
                out_shape=jax.ShapeDtypeStruct((_L, _H), f32),
            )(he, w1b, lp['W2'], lp['W3'], lp['Win'], lp['Wout'])
        else:
            hv = pl.pallas_call(
                _k_encnode, grid=(ngrid,),
                in_specs=[nspec, espec, espec, w128, w128, w128, w128, w128,
                          wi, wo],
                out_specs=nspec,
                out_shape=jax.ShapeDtypeStruct((_L, _H), f32),
            )(hv, he, vn, w1a, w1b, w1c, lp['W2'], lp['W3'],
              lp['Win'], lp['Wout'])
        vn = _sc_gather(hv, nbidx)
        w11a, w11b, w11c = (lp['W11'][:_H], lp['W11'][_H:2 * _H],
                            lp['W11'][2 * _H:])
        he = pl.pallas_call(
            _k_encedge, grid=(ngrid,),
            in_specs=[nspec, espec, espec, w128, w128, w128, w128, w128],
            out_specs=espec,
            out_shape=jax.ShapeDtypeStruct((_E, _H), f32),
        )(hv, he, vn, w11a, w11b, w11c, lp['W12'], lp['W13'])

    seqi = seq.astype(i32)
    rank = jnp.zeros(_L, i32).at[decode_order].set(jnp.arange(_L, dtype=i32))
    bw = jnp.take(rank, nbidx) < jnp.take(rank, qidx)
    seqnb = jnp.take(seqi, nbidx, axis=1)
    ms_idx = jnp.where(bw[None, :], seqnb, 21)
    fv_idx = jnp.where(bw, 21, nbidx + 22)
    ds_idx = jnp.concatenate([ms_idx.reshape(-1), fv_idx]).astype(i32)
    table_ds = jnp.concatenate(
        [params['W_s'].astype(f32), jnp.zeros((1, _H), f32), hv], 0)
    ds = _sc_gather(table_ds, ds_idx)

    boff = jnp.array([[0], [_L]], i32)
    dec_idx = jnp.where(bw[None, :], nbidx[None, :] + boff,
                        2 * _L).reshape(-1).astype(i32)

    hvd = jnp.concatenate([hv, hv], 0)
    zrow = jnp.zeros((1, _H), f32)
    dgrid = 2 * ngrid
    nspec_d = pl.BlockSpec((_TN, _H), lambda g: (g, 0))
    espec_d = pl.BlockSpec((_TE, _H), lambda g: (g, 0))
    espec_s = pl.BlockSpec((_TE, _H), lambda g: (g % ngrid, 0))
    ms_spec = pl.BlockSpec((_TE, _H), lambda g: (g, 0))
    fv_spec = pl.BlockSpec((_TE, _H), lambda g: (2 * ngrid + g % ngrid, 0))
    for lp in params['dec'][:2]:
        mb = _sc_gather(jnp.concatenate([hvd, zrow], 0), dec_idx)
        w1a, w1b, w1c, w1d = (lp['W1'][:_H], lp['W1'][_H:2 * _H],
                              lp['W1'][2 * _H:3 * _H], lp['W1'][3 * _H:])
        hvd = pl.pallas_call(
            _k_dec, grid=(dgrid,),
            in_specs=[nspec_d, espec_s, ms_spec, fv_spec, espec_d,
                      w128, w128, w128, w128, w128, w128, wi, wo],
            out_specs=nspec_d,
            out_shape=jax.ShapeDtypeStruct((2 * _L, _H), f32),
        )(hvd, he, ds, ds, mb, w1a, w1b, w1c, w1d, lp['W2'], lp['W3'],
          lp['Win'], lp['Wout'])

    lp3 = params['dec'][2]
    w1a3, w1b3, w1c3, w1d3 = (lp3['W1'][:_H], lp3['W1'][_H:2 * _H],
                              lp3['W1'][2 * _H:3 * _H], lp3['W1'][3 * _H:])
    t_i = jnp.asarray(token_to_decode, i32)
    st = t_i * _K
    he2 = lax.dynamic_slice(he, (st, 0), (_K, _H))
    he2 = jnp.concatenate([he2, he2], 0)
    ms2 = jnp.concatenate([lax.dynamic_slice(ds, (st, 0), (_K, _H)),
                           lax.dynamic_slice(ds, (_E + st, 0), (_K, _H))], 0)
    fv2 = lax.dynamic_slice(ds, (2 * _E + st, 0), (_K, _H))
    fv2 = jnp.concatenate([fv2, fv2], 0)
    et = lax.dynamic_slice(nbidx, (st,), (_K,))
    bwt = jnp.take(rank, et) < jnp.take(rank, t_i)
    tgt = jnp.concatenate([et, et + _L], 0)
    bw2 = jnp.concatenate([bwt, bwt], 0)
    cols = jnp.arange(2 * _L, dtype=i32)
    oh = ((cols[None, :] == tgt[:, None]) & bw2[:, None]).astype(f32)
    t_arr = t_i.reshape(1)
    bo = params['b_o'].astype(f32).reshape(1, 21)
    probs = pl.pallas_call(
        _k_dec3,
        in_specs=[pl.BlockSpec(memory_space=pltpu.SMEM)] +
                 [pl.BlockSpec(memory_space=pltpu.VMEM)] * 15,
        out_shape=jax.ShapeDtypeStruct((2, 20), f32),
    )(t_arr, hvd, oh, he2, ms2, fv2, w1a3, w1b3, w1c3, w1d3, lp3['W2'],
      lp3['W3'], lp3['Win'], lp3['Wout'], params['W_o'].astype(f32), bo)
    return probs


# speedup vs baseline: 1.1765x; 1.0422x over previous
import functools

import numpy as np
import jax
import jax.numpy as jnp
from jax import lax
from jax.experimental import pallas as pl
from jax.experimental.pallas import tpu as pltpu
from jax.experimental.pallas import tpu_sc as plsc

_L = 512
_K = 48
_H = 128
_E = _L * _K
_NW = 32


def _sc_gather(table, idx, *, n_bufs=8, sub=48):
    n_rows = idx.shape[0]
    v, d = table.shape
    b_per_w = n_rows // _NW
    outer = n_bufs * sub
    n_outer = b_per_w // outer
    assert b_per_w % outer == 0 and sub % 8 == 0
    assert n_rows % (8 * _NW) == 0

    stage = min(v, 256)
    mesh = plsc.VectorSubcoreMesh(core_axis_name="c", subcore_axis_name="s")

    @functools.partial(
        pl.kernel,
        mesh=mesh,
        out_type=jax.ShapeDtypeStruct((n_rows, d), jnp.float32),
        scratch_types=[
            pltpu.VMEM((b_per_w,), jnp.int32),
            pltpu.VMEM((n_bufs, sub, d), jnp.float32),
            pltpu.VMEM((stage, d), jnp.float32),
            pltpu.VMEM_SHARED((v, d), jnp.float32),
            pltpu.SemaphoreType.DMA,
        ],
    )
    def gk(table_hbm, idx_hbm, out_hbm, idx_v, rows_v, stage_v, shared_v,
           sem):
        sid = lax.axis_index("s")
        wid = sid * 2 + lax.axis_index("c")
        base = wid * b_per_w
        pltpu.sync_copy(idx_hbm.at[pl.ds(base, b_per_w)], idx_v)

        @pl.when(sid == 0)
        def _():
            off = 0
            while off < v:
                c = min(stage, v - off)
                pltpu.sync_copy(table_hbm.at[pl.ds(off, c)],
                                stage_v.at[pl.ds(0, c)])
                pltpu.sync_copy(stage_v.at[pl.ds(0, c)],
                                shared_v.at[pl.ds(off, c)])
                off += c

        plsc.subcore_barrier()
        for oc in range(n_outer):
            copies = [
                pltpu.async_copy(
                    shared_v.at[idx_v.at[pl.ds(oc * outer + j * sub, sub)]],
                    rows_v.at[j], sem)
                for j in range(n_bufs)
            ]
            for j in range(n_bufs):
                copies[j].wait()
                pltpu.sync_copy(
                    rows_v.at[j],
                    out_hbm.at[pl.ds(base + oc * outer + j * sub, sub)])

    return gk(table, idx)


def _k_graph(ca_ref, cat_ref, n_ref, c_ref, o_ref, s1_ref, s2_ref,
             pca, pcn, pcc, pco, pcb, eidx_ref, atoms_ref):
    ca = ca_ref[...]
    d2 = jnp.zeros((_L, _L), jnp.float32)
    for c in range(3):
        diff = ca_ref[:, c:c + 1] - cat_ref[c:c + 1, :]
        d2 = d2 + diff * diff
    dist = jnp.sqrt(d2 + 1e-6)
    iota = lax.broadcasted_iota(jnp.int32, (_L, _L), 1).astype(jnp.float32)

    def step(k, carry):
        dmat, eidx = carry
        mn = jnp.min(dmat, axis=1, keepdims=True)
        idxf = jnp.min(jnp.where(dmat <= mn, iota, jnp.float32(_L)),
                       axis=1, keepdims=True)
        kcol = lax.broadcasted_iota(jnp.int32, (_L, _K), 1)
        eidx = jnp.where(kcol == k, idxf.astype(jnp.int32), eidx)
        dmat = jnp.where(iota == idxf, jnp.float32(jnp.inf), dmat)
        return dmat, eidx

    _, eidx = lax.fori_loop(0, _K, step,
                            (dist, jnp.zeros((_L, _K), jnp.int32)))
    eidx_ref[...] = eidx

    nn = n_ref[...]
    cc = c_ref[...]
    oo = o_ref[...]
    b = ca - nn
    c2 = cc - ca
    s1 = s1_ref[...]
    s2 = s2_ref[...]
    a = jnp.dot(b, s1) * jnp.dot(c2, s2) - jnp.dot(b, s2) * jnp.dot(c2, s1)
    cb = -0.58273431 * a + 0.56802827 * b - 0.54067466 * c2 + ca
    atoms_ref[...] = (jnp.dot(ca, pca[...]) + jnp.dot(nn, pcn[...]) +
                      jnp.dot(cc, pcc[...]) + jnp.dot(oo, pco[...]) +
                      jnp.dot(cb, pcb[...]))


def _ln(x):
    m = jnp.mean(x, -1, keepdims=True)
    xm = x - m
    v = jnp.mean(xm * xm, -1, keepdims=True)
    return xm / jnp.sqrt(v + 1e-5)


def _bdot(a, b):
    return jnp.dot(a, b, preferred_element_type=jnp.float32)


_TEB = 512


def _k_feat(nb_ref, qc_ref, pos_ref, a1, a2, p1, p2, a3, b1,
            bpos, wep, wer, be, out_ref):
    q = qc_ref[...]
    n = nb_ref[...]
    q2 = jnp.dot(q * q, a1[...])
    n2 = jnp.dot(n * n, a2[...])
    x = jnp.dot(q, p1[...]) * jnp.dot(n, p2[...])
    cross = jnp.dot(x, a3[...])
    d = jnp.sqrt(jnp.maximum(q2 + n2 - 2.0 * cross, 0.0) + 1e-6)
    db = jnp.dot(d, b1[...])
    lane = lax.broadcasted_iota(jnp.int32, (_TEB, 400), 1)
    mu = 2.0 + (4.0 / 3.0) * (lane % 16).astype(jnp.float32)
    rbf = jnp.exp(-(((db - mu) / 1.25) ** 2))
    ep = pos_ref[...] + bpos[0:1, :]
    pre = _bdot(ep, wep[...]) + _bdot(rbf, wer[...]) + be[0:1, :]
    out_ref[...] = _ln(pre)


_TN = 64
_TE = _TN * _K


def _rmask():
    r = (lax.broadcasted_iota(jnp.int32, (_TE, _TN), 0) // _K ==
         lax.broadcasted_iota(jnp.int32, (_TE, _TN), 1))
    return r.astype(jnp.float32)


def _rtmask():
    r = (lax.broadcasted_iota(jnp.int32, (_TN, _TE), 1) // _K ==
         lax.broadcasted_iota(jnp.int32, (_TN, _TE), 0))
    return r.astype(jnp.float32)


def _node_tail(hv, msum, win, wout, out_ref):
    h1 = _ln(hv + msum / 30.0)
    f = _bdot(jax.nn.gelu(_bdot(h1, win[...])), wout[...])
    out_ref[...] = _ln(h1 + f)


def _k_encnode_first(he_ref, w1b, w2, w3, win, wout, out_ref):
    m = jax.nn.gelu(_bdot(he_ref[...], w1b[...]))
    m = jax.nn.gelu(_bdot(m, w2[...]))
    m = _bdot(m, w3[...])
    msum = _bdot(_rtmask(), m)
    _node_tail(jnp.zeros((_TN, _H), jnp.float32), msum, win, wout, out_ref)


def _k_encnode(hv_ref, he_ref, vn_ref, w1a, w1b, w1c, w2, w3, win, wout,
               out_ref):
    hv = hv_ref[...]
    m1 = (_bdot(_rmask(), _bdot(hv, w1a[...])) +
          _bdot(he_ref[...], w1b[...]) + _bdot(vn_ref[...], w1c[...]))
    m = jax.nn.gelu(m1)
    m = jax.nn.gelu(_bdot(m, w2[...]))
    m = _bdot(m, w3[...])
    msum = _bdot(_rtmask(), m)
    _node_tail(hv, msum, win, wout, out_ref)


def _k_encedge(hv_ref, he_ref, vn_ref, w1a, w1b, w1c, w2, w3, out_ref):
    he = he_ref[...]
    m1 = (_bdot(_rmask(), _bdot(hv_ref[...], w1a[...])) +
          _bdot(he, w1b[...]) + _bdot(vn_ref[...], w1c[...]))
    m = jax.nn.gelu(m1)
    m = jax.nn.gelu(_bdot(m, w2[...]))
    m = _bdot(m, w3[...])
    out_ref[...] = _ln(he + m)


def _k_dec(hvd_ref, he_ref, ms_ref, fv_ref, mb_ref,
           w1a, w1b, w1c, w1d, w2, w3, win, wout, out_ref):
    hvd = hvd_ref[...]
    m1 = (_bdot(_rmask(), _bdot(hvd, w1a[...])) +
          _bdot(he_ref[...], w1b[...]) + _bdot(ms_ref[...], w1c[...]) +
          _bdot(fv_ref[...] + mb_ref[...], w1d[...]))
    m = jax.nn.gelu(m1)
    m = jax.nn.gelu(_bdot(m, w2[...]))
    m = _bdot(m, w3[...])
    msum = _bdot(_rtmask(), m)
    _node_tail(hvd, msum, win, wout, out_ref)


def _k_dec3(t_ref, hvd_ref, oh_ref, he2_ref, ms2_ref, fv2_ref,
            w1a, w1b, w1c, w1d, w2, w3, win, wout, wo_ref, bo_ref, out_ref):
    tt = t_ref[0]
    hvd = hvd_ref[...]
    r0 = hvd_ref[pl.ds(tt, 1), :]
    r1 = hvd_ref[pl.ds(tt + _L, 1), :]
    hvt = jnp.concatenate([r0, r1], 0)
    mb = _bdot(oh_ref[...], hvd)
    rm = (lax.broadcasted_iota(jnp.int32, (2 * _K, 2), 0) // _K ==
          lax.broadcasted_iota(jnp.int32, (2 * _K, 2), 1)).astype(jnp.float32)
    rt = (lax.broadcasted_iota(jnp.int32, (2, 2 * _K), 1) // _K ==
          lax.broadcasted_iota(jnp.int32, (2, 2 * _K), 0)).astype(jnp.float32)
    m1 = (_bdot(rm, _bdot(hvt, w1a[...])) + _bdot(he2_ref[...], w1b[...]) +
          _bdot(ms2_ref[...], w1c[...]) +
          _bdot(fv2_ref[...] + mb, w1d[...]))
    m = jax.nn.gelu(m1)
    m = jax.nn.gelu(_bdot(m, w2[...]))
    m = _bdot(m, w3[...])
    h1 = _ln(hvt + _bdot(rt, m) / 30.0)
    f = _bdot(jax.nn.gelu(_bdot(h1, win[...])), wout[...])
    h3 = _ln(h1 + f)
    lg = jnp.dot(h3, wo_ref[...]) + bo_ref[0:1, :]
    mx = jnp.max(lg, -1, keepdims=True)
    e = jnp.exp(lg - mx)
    p = e / jnp.sum(e, -1, keepdims=True)
    out_ref[...] = p[:, :20]


def _consts():
    a1 = np.zeros((128, 25), np.float32)
    a2 = np.zeros((128, 25), np.float32)
    p1 = np.zeros((128, 75), np.float32)
    p2 = np.zeros((128, 75), np.float32)
    a3 = np.zeros((75, 25), np.float32)
    b1 = np.zeros((25, 400), np.float32)
    for p in range(25):
        a, b = p // 5, p % 5
        for c in range(3):
            a1[3 * a + c, p] = 1.0
            a2[3 * b + c, p] = 1.0
            p1[3 * a + c, 3 * p + c] = 1.0
            p2[3 * b + c, 3 * p + c] = 1.0
            a3[3 * p + c, p] = 1.0
        b1[p, 16 * p:16 * (p + 1)] = 1.0
    s1 = np.zeros((8, 8), np.float32)
    s2 = np.zeros((8, 8), np.float32)
    for j in range(3):
        s1[(j + 1) % 3, j] = 1.0
        s2[(j + 2) % 3, j] = 1.0
    packs = []
    for ai in range(5):
        pk = np.zeros((8, 16), np.float32)
        for c in range(3):
            pk[c, 3 * ai + c] = 1.0
        packs.append(pk)
    return ([jnp.asarray(m) for m in (a1, a2, p1, p2, a3, b1, s1, s2)],
            [jnp.asarray(m) for m in packs])


def _full_spec(shape):
    return pl.BlockSpec(shape, lambda g: tuple(0 for _ in shape))


def kernel(struct, seq, decode_order, token_to_decode, params):
    f32, i32 = jnp.float32, jnp.int32
    struct = struct.astype(f32)
    nn, ca, cc, oo = (struct[:, 0, :], struct[:, 1, :],
                      struct[:, 2, :], struct[:, 3, :])
    pad8 = lambda x: jnp.pad(x, ((0, 0), (0, 5)))
    ca8, n8, c8, o8 = pad8(ca), pad8(nn), pad8(cc), pad8(oo)
    cat = ca8.T
    (a1, a2, p1, p2, a3, b1, s1, s2), packs = _consts()

    eidx, atoms16 = pl.pallas_call(
        _k_graph,
        out_shape=[jax.ShapeDtypeStruct((_L, _K), i32),
                   jax.ShapeDtypeStruct((_L, 16), f32)],
    )(ca8, cat, n8, c8, o8, s1, s2, *packs)

    nbidx = eidx.reshape(-1).astype(i32)
    qidx = jnp.repeat(jnp.arange(_L, dtype=i32), _K)
    posidx = jnp.clip(qidx - nbidx + 32, 0, 64) + _L
    feat_idx = jnp.concatenate([nbidx, qidx, posidx])

    wpos = params['W_pos'].astype(f32)
    table_feat = jnp.pad(jnp.concatenate([atoms16, wpos], 0),
                         ((0, 0), (0, 112)))
    gf = _sc_gather(table_feat, feat_idx)

    bpos = jnp.tile(jnp.pad(params['b_pos'].astype(f32), (0, 112))[None, :],
                    (8, 1))
    be = jnp.tile(params['b_e'].astype(f32)[None, :], (8, 1))
    wep = jnp.pad(params['W_e'][:16].astype(f32), ((0, 112), (0, 0)))
    wer = params['W_e'][16:].astype(f32)
    n_ft = _E // _TEB
    he = pl.pallas_call(
        _k_feat,
        grid=(n_ft,),
        in_specs=[pl.BlockSpec((_TEB, _H), lambda g: (g, 0)),
                  pl.BlockSpec((_TEB, _H), lambda g: (48 + g, 0)),
                  pl.BlockSpec((_TEB, _H), lambda g: (96 + g, 0)),
                  _full_spec((_H, 25)), _full_spec((_H, 25)),
                  _full_spec((_H, 75)), _full_spec((_H, 75)),
                  _full_spec((75, 25)), _full_spec((25, 400)),
                  _full_spec((8, _H)), _full_spec((_H, _H)),
                  _full_spec((400, _H)), _full_spec((8, _H))],
        out_specs=pl.BlockSpec((_TEB, _H), lambda g: (g, 0)),
        out_shape=jax.ShapeDtypeStruct((_E, _H), f32),
    )(gf, gf, gf, a1, a2, p1, p2, a3, b1, bpos, wep, wer, be)

    ngrid = _L // _TN
    nspec = pl.BlockSpec((_TN, _H), lambda g: (g, 0))
    espec = pl.BlockSpec((_TE, _H), lambda g: (g, 0))
    w128 = _full_spec((_H, _H))
    wi = _full_spec((_H, 4 * _H))
    wo = _full_spec((4 * _H, _H))

    hv = None
    vn = None
    for li, lp in enumerate(params['enc']):
        w1a, w1b, w1c = (lp['W1'][:_H], lp['W1'][_H:2 * _H],
                         lp['W1'][2 * _H:])
        if li == 0:
            hv = pl.pallas_call(
                _k_encnode_first, grid=(ngrid,),
                in_specs=[espec, w128, w128, w128, wi, wo],
                out_specs=nspec,
                out_shape=jax.ShapeDtypeStruct((_L, _H), f32),
            )(he, w1b, lp['W2'], lp['W3'], lp['Win'], lp['Wout'])
        else:
            hv = pl.pallas_call(
                _k_encnode, grid=(ngrid,),
                in_specs=[nspec, espec, espec, w128, w128, w128, w128, w128,
                          wi, wo],
                out_specs=nspec,
                out_shape=jax.ShapeDtypeStruct((_L, _H), f32),
            )(hv, he, vn, w1a, w1b, w1c, lp['W2'], lp['W3'],
              lp['Win'], lp['Wout'])
        vn = _sc_gather(hv, nbidx)
        w11a, w11b, w11c = (lp['W11'][:_H], lp['W11'][_H:2 * _H],
                            lp['W11'][2 * _H:])
        he = pl.pallas_call(
            _k_encedge, grid=(ngrid,),
            in_specs=[nspec, espec, espec, w128, w128, w128, w128, w128],
            out_specs=espec,
            out_shape=jax.ShapeDtypeStruct((_E, _H), f32),
        )(hv, he, vn, w11a, w11b, w11c, lp['W12'], lp['W13'])

    seqi = seq.astype(i32)
    rank = jnp.zeros(_L, i32).at[decode_order].set(jnp.arange(_L, dtype=i32))
    bw = jnp.take(rank, nbidx) < jnp.take(rank, qidx)
    seqnb = jnp.take(seqi, nbidx, axis=1)
    ms_idx = jnp.where(bw[None, :], seqnb, 21)
    fv_idx = jnp.where(bw, 21, nbidx + 22)
    mb1_idx = jnp.where(bw, nbidx + 22, 21)
    ds_idx = jnp.concatenate([ms_idx.reshape(-1), fv_idx,
                              mb1_idx]).astype(i32)
    table_ds = jnp.concatenate(
        [params['W_s'].astype(f32), jnp.zeros((1, _H), f32), hv], 0)
    ds = _sc_gather(table_ds, ds_idx)

    boff = jnp.array([[0], [_L]], i32)
    dec_idx = jnp.where(bw[None, :], nbidx[None, :] + boff,
                        2 * _L).reshape(-1).astype(i32)

    hvd = jnp.concatenate([hv, hv], 0)
    zrow = jnp.zeros((1, _H), f32)
    dgrid = 2 * ngrid
    nspec_d = pl.BlockSpec((_TN, _H), lambda g: (g, 0))
    espec_d = pl.BlockSpec((_TE, _H), lambda g: (g, 0))
    espec_s = pl.BlockSpec((_TE, _H), lambda g: (g % ngrid, 0))
    ms_spec = pl.BlockSpec((_TE, _H), lambda g: (g, 0))
    fv_spec = pl.BlockSpec((_TE, _H), lambda g: (2 * ngrid + g % ngrid, 0))
    mb1_spec = pl.BlockSpec((_TE, _H), lambda g: (3 * ngrid + g % ngrid, 0))

    def dec_layer(hvd, mb_arr, mb_spec, lp):
        w1a, w1b, w1c, w1d = (lp['W1'][:_H], lp['W1'][_H:2 * _H],
                              lp['W1'][2 * _H:3 * _H], lp['W1'][3 * _H:])
        return pl.pallas_call(
            _k_dec, grid=(dgrid,),
            in_specs=[nspec_d, espec_s, ms_spec, fv_spec, mb_spec,
                      w128, w128, w128, w128, w128, w128, wi, wo],
            out_specs=nspec_d,
            out_shape=jax.ShapeDtypeStruct((2 * _L, _H), f32),
        )(hvd, he, ds, ds, mb_arr, w1a, w1b, w1c, w1d, lp['W2'], lp['W3'],
          lp['Win'], lp['Wout'])

    hvd = dec_layer(hvd, ds, mb1_spec, params['dec'][0])
    mb2 = _sc_gather(jnp.concatenate([hvd, zrow], 0), dec_idx)
    hvd = dec_layer(hvd, mb2, espec_d, params['dec'][1])

    lp3 = params['dec'][2]
    w1a3, w1b3, w1c3, w1d3 = (lp3['W1'][:_H], lp3['W1'][_H:2 * _H],
                              lp3['W1'][2 * _H:3 * _H], lp3['W1'][3 * _H:])
    t_i = jnp.asarray(token_to_decode, i32)
    st = t_i * _K
    he2 = lax.dynamic_slice(he, (st, 0), (_K, _H))
    he2 = jnp.concatenate([he2, he2], 0)
    ms2 = jnp.concatenate([lax.dynamic_slice(ds, (st, 0), (_K, _H)),
                           lax.dynamic_slice(ds, (_E + st, 0), (_K, _H))], 0)
    fv2 = lax.dynamic_slice(ds, (2 * _E + st, 0), (_K, _H))
    fv2 = jnp.concatenate([fv2, fv2], 0)
    et = lax.dynamic_slice(nbidx, (st,), (_K,))
    bwt = jnp.take(rank, et) < jnp.take(rank, t_i)
    tgt = jnp.concatenate([et, et + _L], 0)
    bw2 = jnp.concatenate([bwt, bwt], 0)
    cols = jnp.arange(2 * _L, dtype=i32)
    oh = ((cols[None, :] == tgt[:, None]) & bw2[:, None]).astype(f32)
    t_arr = t_i.reshape(1)
    bo = params['b_o'].astype(f32).reshape(1, 21)
    probs = pl.pallas_call(
        _k_dec3,
        in_specs=[pl.BlockSpec(memory_space=pltpu.SMEM)] +
                 [pl.BlockSpec(memory_space=pltpu.VMEM)] * 15,
        out_shape=jax.ShapeDtypeStruct((2, 20), f32),
    )(t_arr, hvd, oh, he2, ms2, fv2, w1a3, w1b3, w1c3, w1d3, lp3['W2'],
      lp3['W3'], lp3['Win'], lp3['Wout'], params['W_o'].astype(f32), bo)
    return probs
